# Initial kernel scaffold; baseline (speedup 1.0000x reference)
import jax
import jax.numpy as jnp
from jax.experimental import pallas as pl


def kernel(h, x, vel, edge_attr, params, edges):
    raise NotImplementedError("write your pallas kernel here")


# speedup vs baseline: 1.8094x; 1.8094x over previous
import functools

import jax
import jax.numpy as jnp
from jax import lax
from jax.experimental import pallas as pl
from jax.experimental.pallas import tpu as pltpu
from jax.experimental.pallas import tpu_sc as plsc

N = 10000
E = 320000
HID = 128
HALF = 64
N_LAYERS = 4
N_POINTS = 5

NC = 2
NS = 16
NW = NC * NS
CK = 128
NCHUNK = E // CK
ITERS = -(-NCHUNK // NW)
RPT8 = 624
NTAIL = N - NS * RPT8

DG = 256
DS = 128
XO = 128
VO = 131

BE = 1280
BN = 2000


def _sc_mesh():
    return plsc.VectorSubcoreMesh(core_axis_name="c", subcore_axis_name="s",
                                  num_cores=NC, num_subcores=NS)


def _silu(v):
    return v * (1.0 / (1.0 + jnp.exp(-v)))


def _cross(ax, ay, az, bx, by, bz):
    return (ay * bz - az * by, az * bx - ax * bz, ax * by - ay * bx)


def _frame_cols(xr, xl):
    dx, dy, dz = xr[0] - xl[0], xr[1] - xl[1], xr[2] - xl[2]
    radial = dx * dx + dy * dy + dz * dz
    nrm = jnp.sqrt(radial) + 1.0
    cd = (dx / nrm, dy / nrm, dz / nrm)
    cx, cy, cz = _cross(xr[0], xr[1], xr[2], xl[0], xl[1], xl[2])
    cn = jnp.sqrt(cx * cx + cy * cy + cz * cz) + 1.0
    cc = (cx / cn, cy / cn, cz / cn)
    cv = _cross(cd[0], cd[1], cd[2], cc[0], cc[1], cc[2])
    return cd, cc, cv, radial


@functools.lru_cache(maxsize=None)
def _make_gather2():
    @functools.partial(
        pl.kernel,
        out_type=(jax.ShapeDtypeStruct((E, DG), jnp.float32),
                  jax.ShapeDtypeStruct((E, DG), jnp.float32)),
        mesh=_sc_mesh(),
        scratch_types=[
            pltpu.VMEM((CK,), jnp.int32),
            pltpu.VMEM((CK,), jnp.int32),
            pltpu.VMEM((CK, DG), jnp.float32),
            pltpu.VMEM((CK, DG), jnp.float32),
            pltpu.SemaphoreType.DMA,
            pltpu.SemaphoreType.DMA,
        ],
    )
    def k(ta, tb, ia, ib, oa, ob, idx_a, idx_b, buf_a, buf_b, sem_a, sem_b):
        wid = lax.axis_index("s") * NC + lax.axis_index("c")

        def body(i, carry):
            cid = wid + i * NW

            @pl.when(cid < NCHUNK)
            def _():
                base = cid * CK
                pltpu.sync_copy(ia.at[pl.ds(base, CK)], idx_a)
                pltpu.sync_copy(ib.at[pl.ds(base, CK)], idx_b)
                ca = pltpu.async_copy(ta.at[idx_a], buf_a, sem_a)
                cb = pltpu.async_copy(tb.at[idx_b], buf_b, sem_b)
                ca.wait()
                cb.wait()
                pltpu.sync_copy(buf_a, oa.at[pl.ds(base, CK)])
                pltpu.sync_copy(buf_b, ob.at[pl.ds(base, CK)])

            return carry

        lax.fori_loop(0, ITERS, body, 0)

    return k


@functools.lru_cache(maxsize=None)
def _make_scatter():
    @functools.partial(
        pl.kernel,
        out_type=jax.ShapeDtypeStruct((NC, N, DS), jnp.float32),
        mesh=_sc_mesh(),
        scratch_types=[
            pltpu.VMEM((CK,), jnp.int32),
            pltpu.VMEM((CK, DS), jnp.float32),
            pltpu.VMEM_SHARED((N, DS), jnp.float32),
        ],
    )
    def k(data, idx_hbm, zeros, out, idx_v, buf, acc):
        c = lax.axis_index("c")
        s = lax.axis_index("s")
        wid = s * NC + c
        ib = s * RPT8
        pltpu.sync_copy(zeros.at[pl.ds(ib, RPT8)], acc.at[pl.ds(ib, RPT8)])

        @pl.when(s == NS - 1)
        def _init_tail():
            pltpu.sync_copy(zeros.at[pl.ds(NS * RPT8, NTAIL)],
                            acc.at[pl.ds(NS * RPT8, NTAIL)])

        plsc.subcore_barrier()

        def body(i, carry):
            cid = wid + i * NW

            @pl.when(cid < NCHUNK)
            def _():
                base = cid * CK
                pltpu.sync_copy(idx_hbm.at[pl.ds(base, CK)], idx_v)
                pltpu.sync_copy(data.at[pl.ds(base, CK)], buf)
                pltpu.sync_copy(buf, acc.at[idx_v], add=True)

            return carry

        lax.fori_loop(0, ITERS, body, 0)
        plsc.subcore_barrier()
        pltpu.sync_copy(acc.at[pl.ds(ib, RPT8)], out.at[c, pl.ds(ib, RPT8)])

        @pl.when(s == NS - 1)
        def _drain_tail():
            pltpu.sync_copy(acc.at[pl.ds(NS * RPT8, NTAIL)],
                            out.at[c, pl.ds(NS * RPT8, NTAIL)])

    return k


def _gather2(ta, tb, ia, ib):
    return _make_gather2()(ta, tb, ia, ib)


def _scatter(data, idx, zeros):
    return _make_scatter()(data, idx, zeros)


def _full_spec(shape):
    nd = len(shape)
    return pl.BlockSpec(shape, lambda i, _n=nd: (0,) * _n)


def _center_body(x15_ref, m1_ref, m2_ref, xc15_ref, cent15_ref):
    x15 = x15_ref[...]
    cent = jnp.dot(jnp.dot(x15, m1_ref[...], preferred_element_type=jnp.float32),
                   m2_ref[...], preferred_element_type=jnp.float32)
    cent15_ref[...] = cent
    xc15_ref[...] = x15 - cent


def _center(x15, m1, m2):
    g = x15.shape[0]
    return pl.pallas_call(
        _center_body,
        grid=(1,),
        in_specs=[_full_spec((g, 15)), _full_spec((15, 3)), _full_spec((3, 15))],
        out_specs=[_full_spec((g, 15)), _full_spec((g, 15))],
        out_shape=[jax.ShapeDtypeStruct((g, 15), jnp.float32),
                   jax.ShapeDtypeStruct((g, 15), jnp.float32)],
    )(x15, m1, m2)


def _embed_body(h0_ref, xc_ref, vel_ref, wemb_ref, bemb_ref, wr_ref, wc_ref,
                h_ref, rtab_ref, ctab_ref):
    h = jnp.dot(h0_ref[...], wemb_ref[...], preferred_element_type=jnp.float32) \
        + bemb_ref[...]
    h_ref[...] = h
    xc = xc_ref[...]
    vel = vel_ref[...]
    z = jnp.zeros((h.shape[0], DG - VO - 3), jnp.float32)
    rtab_ref[...] = jnp.concatenate(
        [jnp.dot(h, wr_ref[...], preferred_element_type=jnp.float32), xc, vel, z],
        axis=1)
    ctab_ref[...] = jnp.concatenate(
        [jnp.dot(h, wc_ref[...], preferred_element_type=jnp.float32), xc, vel, z],
        axis=1)


def _embed(h0, xc, vel, wemb, bemb, wr, wc):
    grid = N // BN
    return pl.pallas_call(
        _embed_body,
        grid=(grid,),
        in_specs=[
            pl.BlockSpec((BN, HID), lambda i: (i, 0)),
            pl.BlockSpec((BN, 3), lambda i: (i, 0)),
            pl.BlockSpec((BN, 3), lambda i: (i, 0)),
            _full_spec((HID, HID)),
            _full_spec((1, HID)),
            _full_spec((HID, HID)),
            _full_spec((HID, HID)),
        ],
        out_specs=[
            pl.BlockSpec((BN, HID), lambda i: (i, 0)),
            pl.BlockSpec((BN, DG), lambda i: (i, 0)),
            pl.BlockSpec((BN, DG), lambda i: (i, 0)),
        ],
        out_shape=[jax.ShapeDtypeStruct((N, HID), jnp.float32),
                   jax.ShapeDtypeStruct((N, DG), jnp.float32),
                   jax.ShapeDtypeStruct((N, DG), jnp.float32)],
    )(h0, xc, vel, wemb, bemb, wr, wc)


def _cols3(arr, off):
    return (arr[:, off:off + 1], arr[:, off + 1:off + 2], arr[:, off + 2:off + 3])


def _ef_body(grv_ref, gcv_ref, ea_ref, wf1_ref, bf1_ref, wf2_ref, bf2_ref, ef_ref):
    grv = grv_ref[...]
    gcv = gcv_ref[...]
    xr = _cols3(grv, XO)
    vr = _cols3(grv, VO)
    xl = _cols3(gcv, XO)
    vc = _cols3(gcv, VO)
    cd, cc, cv, _ = _frame_cols(xr, xl)

    def proj(v):
        return (cd[0] * v[0] + cd[1] * v[1] + cd[2] * v[2],
                cc[0] * v[0] + cc[1] * v[1] + cc[2] * v[2],
                cv[0] * v[0] + cv[1] * v[1] + cv[2] * v[2])

    ci = proj(xr)
    cj = proj(xl)
    vi = proj(vr)
    vj = proj(vc)
    ni = jnp.sqrt(ci[0] * ci[0] + ci[1] * ci[1] + ci[2] * ci[2])
    nj = jnp.sqrt(cj[0] * cj[0] + cj[1] * cj[1] + cj[2] * cj[2])
    cos = (ci[0] * cj[0] + ci[1] * cj[1] + ci[2] * cj[2]) / (ni + 1e-05) / (nj + 1e-05)
    sin = jnp.sqrt(jnp.clip(1.0 - cos * cos, 0.0, None))
    feat = jnp.concatenate(
        [ea_ref[...], sin, cos, ci[0], ci[1], ci[2], cj[0], cj[1], cj[2],
         vi[0], vi[1], vi[2], vj[0], vj[1], vj[2]], axis=1)
    e1 = _silu(jnp.dot(feat, wf1_ref[...], preferred_element_type=jnp.float32)
               + bf1_ref[...])
    ef_ref[...] = _silu(jnp.dot(e1, wf2_ref[...], preferred_element_type=jnp.float32)
                        + bf2_ref[...])


def _ef_mlp(grv, gcv, ea, wf1, bf1, wf2, bf2):
    grid = E // BE
    return pl.pallas_call(
        _ef_body,
        grid=(grid,),
        in_specs=[
            pl.BlockSpec((BE, DG), lambda i: (i, 0)),
            pl.BlockSpec((BE, DG), lambda i: (i, 0)),
            pl.BlockSpec((BE, 2), lambda i: (i, 0)),
            _full_spec((16, HALF)),
            _full_spec((1, HALF)),
            _full_spec((HALF, HALF)),
            _full_spec((1, HALF)),
        ],
        out_specs=pl.BlockSpec((BE, HALF), lambda i: (i, 0)),
        out_shape=jax.ShapeDtypeStruct((E, HALF), jnp.float32),
    )(grv, gcv, ea, wf1, bf1, wf2, bf2)


def _edge_body(last, gr_ref, gc_ref, ef_ref, wef_ref, wrad_ref, b1_ref,
               w2_ref, b2_ref, wc1_ref, bc1_ref, wc2_ref, *out_refs):
    gr = gr_ref[...]
    gc = gc_ref[...]
    hs = gr[:, :HID] + gc[:, :HID]
    xr = _cols3(gr, XO)
    xl = _cols3(gc, XO)
    cd, cc, cv, radial = _frame_cols(xr, xl)
    z1 = hs + radial * wrad_ref[...] + b1_ref[...] \
        + jnp.dot(ef_ref[...], wef_ref[...], preferred_element_type=jnp.float32)
    a1 = _silu(z1)
    m = _silu(jnp.dot(a1, w2_ref[...], preferred_element_type=jnp.float32)
              + b2_ref[...])
    c1 = _silu(jnp.dot(m, wc1_ref[...], preferred_element_type=jnp.float32)
               + bc1_ref[...])
    coff = jnp.dot(c1, wc2_ref[...], preferred_element_type=jnp.float32)
    c0, c1_, c2 = coff[:, 0:1], coff[:, 1:2], coff[:, 2:3]
    tx = jnp.clip(cd[0] * c0 + cc[0] * c1_ + cv[0] * c2, -100.0, 100.0)
    ty = jnp.clip(cd[1] * c0 + cc[1] * c1_ + cv[1] * c2, -100.0, 100.0)
    tz = jnp.clip(cd[2] * c0 + cc[2] * c1_ + cv[2] * c2, -100.0, 100.0)
    one = jnp.ones_like(tx)
    pad = jnp.zeros((gr.shape[0], DS - 4), jnp.float32)
    tvec = jnp.concatenate([tx, ty, tz, one, pad], axis=1)
    if last:
        out_refs[0][...] = tvec
    else:
        out_refs[0][...] = m
        out_refs[1][...] = tvec


def _edge_mlp(last, gr, gc, ef, wef, wrad, b1, w2, b2, wc1, bc1, wc2):
    grid = E // BE
    if last:
        out_specs = [pl.BlockSpec((BE, DS), lambda i: (i, 0))]
        out_shape = [jax.ShapeDtypeStruct((E, DS), jnp.float32)]
    else:
        out_specs = [pl.BlockSpec((BE, HID), lambda i: (i, 0)),
                     pl.BlockSpec((BE, DS), lambda i: (i, 0))]
        out_shape = [jax.ShapeDtypeStruct((E, HID), jnp.float32),
                     jax.ShapeDtypeStruct((E, DS), jnp.float32)]
    return pl.pallas_call(
        functools.partial(_edge_body, last),
        grid=(grid,),
        in_specs=[
            pl.BlockSpec((BE, DG), lambda i: (i, 0)),
            pl.BlockSpec((BE, DG), lambda i: (i, 0)),
            pl.BlockSpec((BE, HALF), lambda i: (i, 0)),
            _full_spec((HALF, HID)),
            _full_spec((1, HID)),
            _full_spec((1, HID)),
            _full_spec((HID, HID)),
            _full_spec((1, HID)),
            _full_spec((HID, HID)),
            _full_spec((1, HID)),
            _full_spec((HID, 3)),
        ],
        out_specs=out_specs,
        out_shape=out_shape,
    )(gr, gc, ef, wef, wrad, b1, w2, b2, wc1, bc1, wc2)


def _node_mid_body(h_ref, mpm_ref, mpt_ref, xc_ref, vel_ref,
                   wv1_ref, bv1_ref, wv2_ref, bv2_ref,
                   wn1h_ref, wn1g_ref, bn1_ref, wn2_ref, bn2_ref,
                   wrn_ref, wcn_ref,
                   hn_ref, xcn_ref, rtab_ref, ctab_ref):
    h = h_ref[...]
    mpm = mpm_ref[...]
    mpt = mpt_ref[...]
    hag = mpm[0] + mpm[1]
    tsum = mpt[0] + mpt[1]
    agg = tsum[:, 0:3]
    cnt = tsum[:, 3:4]
    xc = xc_ref[...] + agg / jnp.maximum(cnt, 1.0)
    vmul = jnp.dot(_silu(jnp.dot(h, wv1_ref[...], preferred_element_type=jnp.float32)
                         + bv1_ref[...]),
                   wv2_ref[...], preferred_element_type=jnp.float32) + bv2_ref[...]
    xc = xc + vmul * vel_ref[...]
    t = _silu(jnp.dot(h, wn1h_ref[...], preferred_element_type=jnp.float32)
              + jnp.dot(hag, wn1g_ref[...], preferred_element_type=jnp.float32)
              + bn1_ref[...])
    hn = h + jnp.dot(t, wn2_ref[...], preferred_element_type=jnp.float32) + bn2_ref[...]
    hn_ref[...] = hn
    xcn_ref[...] = xc
    z = jnp.zeros((h.shape[0], DG - XO - 3), jnp.float32)
    rtab_ref[...] = jnp.concatenate(
        [jnp.dot(hn, wrn_ref[...], preferred_element_type=jnp.float32), xc, z], axis=1)
    ctab_ref[...] = jnp.concatenate(
        [jnp.dot(hn, wcn_ref[...], preferred_element_type=jnp.float32), xc, z], axis=1)


def _node_mid(h, mpm, mpt, xc, vel, wv1, bv1, wv2, bv2,
              wn1h, wn1g, bn1, wn2, bn2, wrn, wcn):
    grid = N // BN
    return pl.pallas_call(
        _node_mid_body,
        grid=(grid,),
        in_specs=[
            pl.BlockSpec((BN, HID), lambda i: (i, 0)),
            pl.BlockSpec((NC, BN, HID), lambda i: (0, i, 0)),
            pl.BlockSpec((NC, BN, DS), lambda i: (0, i, 0)),
            pl.BlockSpec((BN, 3), lambda i: (i, 0)),
            pl.BlockSpec((BN, 3), lambda i: (i, 0)),
            _full_spec((HID, HID)),
            _full_spec((1, HID)),
            _full_spec((HID, 1)),
            _full_spec((1, 1)),
            _full_spec((HID, HID)),
            _full_spec((HID, HID)),
            _full_spec((1, HID)),
            _full_spec((HID, HID)),
            _full_spec((1, HID)),
            _full_spec((HID, HID)),
            _full_spec((HID, HID)),
        ],
        out_specs=[
            pl.BlockSpec((BN, HID), lambda i: (i, 0)),
            pl.BlockSpec((BN, 3), lambda i: (i, 0)),
            pl.BlockSpec((BN, DG), lambda i: (i, 0)),
            pl.BlockSpec((BN, DG), lambda i: (i, 0)),
        ],
        out_shape=[jax.ShapeDtypeStruct((N, HID), jnp.float32),
                   jax.ShapeDtypeStruct((N, 3), jnp.float32),
                   jax.ShapeDtypeStruct((N, DG), jnp.float32),
                   jax.ShapeDtypeStruct((N, DG), jnp.float32)],
    )(h, mpm, mpt, xc, vel, wv1, bv1, wv2, bv2, wn1h, wn1g, bn1, wn2, bn2, wrn, wcn)


def _node_last_body(h_ref, mpt_ref, xc_ref, vel_ref, cent_ref,
                    wv1_ref, bv1_ref, wv2_ref, bv2_ref, out_ref):
    h = h_ref[...]
    mpt = mpt_ref[...]
    tsum = mpt[0] + mpt[1]
    agg = tsum[:, 0:3]
    cnt = tsum[:, 3:4]
    xc = xc_ref[...] + agg / jnp.maximum(cnt, 1.0)
    vmul = jnp.dot(_silu(jnp.dot(h, wv1_ref[...], preferred_element_type=jnp.float32)
                         + bv1_ref[...]),
                   wv2_ref[...], preferred_element_type=jnp.float32) + bv2_ref[...]
    xc = xc + vmul * vel_ref[...]
    out_ref[...] = xc + cent_ref[...]


def _node_last(h, mpt, xc, vel, cent, wv1, bv1, wv2, bv2):
    grid = N // BN
    return pl.pallas_call(
        _node_last_body,
        grid=(grid,),
        in_specs=[
            pl.BlockSpec((BN, HID), lambda i: (i, 0)),
            pl.BlockSpec((NC, BN, DS), lambda i: (0, i, 0)),
            pl.BlockSpec((BN, 3), lambda i: (i, 0)),
            pl.BlockSpec((BN, 3), lambda i: (i, 0)),
            pl.BlockSpec((BN, 3), lambda i: (i, 0)),
            _full_spec((HID, HID)),
            _full_spec((1, HID)),
            _full_spec((HID, 1)),
            _full_spec((1, 1)),
        ],
        out_specs=pl.BlockSpec((BN, 3), lambda i: (i, 0)),
        out_shape=jax.ShapeDtypeStruct((N, 3), jnp.float32),
    )(h, mpt, xc, vel, cent, wv1, bv1, wv2, bv2)


def kernel(h, x, vel, edge_attr, params, edges):
    row = edges[0]
    col = edges[1]
    f32 = jnp.float32

    m1 = jnp.zeros((15, 3), f32).at[jnp.arange(15), jnp.arange(15) % 3].set(0.2)
    m2 = jnp.zeros((3, 15), f32).at[jnp.arange(15) % 3, jnp.arange(15)].set(1.0)

    x15 = x.reshape(N // N_POINTS, 15)
    xc15, cent15 = _center(x15, m1, m2)
    xc = xc15.reshape(N, 3)
    cent = cent15.reshape(N, 3)

    we1 = params['We1']
    wr0 = we1[0, 0:HID, :]
    wc0 = we1[0, HID:2 * HID, :]
    hh, rtab, ctab = _embed(
        h, xc, vel, params['Wemb'], params['bemb'].reshape(1, HID), wr0, wc0)

    zeros_s = jnp.zeros((N, DS), f32)
    ef = None

    for l in range(N_LAYERS):
        last = l == N_LAYERS - 1
        gr, gc = _gather2(rtab, ctab, row, col)
        if l == 0:
            ef = _ef_mlp(gr, gc, edge_attr,
                         params['Wf1'], params['bf1'].reshape(1, HALF),
                         params['Wf2'], params['bf2'].reshape(1, HALF))
        ed = _edge_mlp(
            last, gr, gc, ef,
            we1[l, 2 * HID + 1:, :], we1[l, 2 * HID:2 * HID + 1, :],
            params['be1'][l].reshape(1, HID),
            params['We2'][l], params['be2'][l].reshape(1, HID),
            params['Wc1'][l], params['bc1'][l].reshape(1, HID),
            params['Wc2'][l])
        if last:
            mpt = _scatter(ed[0], row, zeros_s)
            out = _node_last(
                hh, mpt, xc, vel, cent,
                params['Wv1'][l], params['bv1'][l].reshape(1, HID),
                params['Wv2'][l], params['bv2'][l].reshape(1, 1))
        else:
            mpm = _scatter(ed[0], row, zeros_s)
            mpt = _scatter(ed[1], row, zeros_s)
            hh, xc, rtab, ctab = _node_mid(
                hh, mpm, mpt, xc, vel,
                params['Wv1'][l], params['bv1'][l].reshape(1, HID),
                params['Wv2'][l], params['bv2'][l].reshape(1, 1),
                params['Wn1'][l][0:HID, :], params['Wn1'][l][HID:, :],
                params['bn1'][l].reshape(1, HID),
                params['Wn2'][l], params['bn2'][l].reshape(1, HID),
                we1[l + 1, 0:HID, :], we1[l + 1, HID:2 * HID, :])
    return out


# speedup vs baseline: 1.9344x; 1.0691x over previous
import functools

import jax
import jax.numpy as jnp
from jax import lax
from jax.experimental import pallas as pl
from jax.experimental.pallas import tpu as pltpu
from jax.experimental.pallas import tpu_sc as plsc

N = 10000
E = 320000
HID = 128
HALF = 64
N_LAYERS = 4
N_POINTS = 5

NC = 2
NS = 16
NW = NC * NS
CK = 128
NCHUNK = E // CK
ITERS = -(-NCHUNK // NW)
RPT8 = 624
NTAIL = N - NS * RPT8

DG = 256
DS = 128
XO = 128
VO = 131

BE = 1280
BN = 2000


def _sc_mesh():
    return plsc.VectorSubcoreMesh(core_axis_name="c", subcore_axis_name="s",
                                  num_cores=NC, num_subcores=NS)


def _silu(v):
    return v * (1.0 / (1.0 + jnp.exp(-v)))


def _cross(ax, ay, az, bx, by, bz):
    return (ay * bz - az * by, az * bx - ax * bz, ax * by - ay * bx)


def _frame_cols(xr, xl):
    dx, dy, dz = xr[0] - xl[0], xr[1] - xl[1], xr[2] - xl[2]
    radial = dx * dx + dy * dy + dz * dz
    nrm = jnp.sqrt(radial) + 1.0
    cd = (dx / nrm, dy / nrm, dz / nrm)
    cx, cy, cz = _cross(xr[0], xr[1], xr[2], xl[0], xl[1], xl[2])
    cn = jnp.sqrt(cx * cx + cy * cy + cz * cz) + 1.0
    cc = (cx / cn, cy / cn, cz / cn)
    cv = _cross(cd[0], cd[1], cd[2], cc[0], cc[1], cc[2])
    return cd, cc, cv, radial


@functools.lru_cache(maxsize=None)
def _make_gather2():
    @functools.partial(
        pl.kernel,
        out_type=(jax.ShapeDtypeStruct((E, DG), jnp.float32),
                  jax.ShapeDtypeStruct((E, DG), jnp.float32)),
        mesh=_sc_mesh(),
        scratch_types=[
            pltpu.VMEM((CK,), jnp.int32),
            pltpu.VMEM((CK,), jnp.int32),
            pltpu.VMEM((CK, DG), jnp.float32),
            pltpu.VMEM((CK, DG), jnp.float32),
            pltpu.SemaphoreType.DMA,
            pltpu.SemaphoreType.DMA,
        ],
    )
    def k(ta, tb, ia, ib, oa, ob, idx_a, idx_b, buf_a, buf_b, sem_a, sem_b):
        wid = lax.axis_index("s") * NC + lax.axis_index("c")

        def body(i, carry):
            cid = wid + i * NW

            @pl.when(cid < NCHUNK)
            def _():
                base = cid * CK
                pltpu.sync_copy(ia.at[pl.ds(base, CK)], idx_a)
                pltpu.sync_copy(ib.at[pl.ds(base, CK)], idx_b)
                ca = pltpu.async_copy(ta.at[idx_a], buf_a, sem_a)
                cb = pltpu.async_copy(tb.at[idx_b], buf_b, sem_b)
                ca.wait()
                cb.wait()
                pltpu.sync_copy(buf_a, oa.at[pl.ds(base, CK)])
                pltpu.sync_copy(buf_b, ob.at[pl.ds(base, CK)])

            return carry

        lax.fori_loop(0, ITERS, body, 0)

    return k


@functools.lru_cache(maxsize=None)
def _make_scatter():
    @functools.partial(
        pl.kernel,
        out_type=jax.ShapeDtypeStruct((NC, N, DS), jnp.float32),
        mesh=_sc_mesh(),
        scratch_types=[
            pltpu.VMEM((CK,), jnp.int32),
            pltpu.VMEM((CK, DS), jnp.float32),
            pltpu.VMEM_SHARED((N, DS), jnp.float32),
        ],
    )
    def k(data, idx_hbm, zeros, out, idx_v, buf, acc):
        c = lax.axis_index("c")
        s = lax.axis_index("s")
        wid = s * NC + c
        ib = s * RPT8
        pltpu.sync_copy(zeros.at[pl.ds(ib, RPT8)], acc.at[pl.ds(ib, RPT8)])

        @pl.when(s == NS - 1)
        def _init_tail():
            pltpu.sync_copy(zeros.at[pl.ds(NS * RPT8, NTAIL)],
                            acc.at[pl.ds(NS * RPT8, NTAIL)])

        plsc.subcore_barrier()

        def body(i, carry):
            cid = wid + i * NW

            @pl.when(cid < NCHUNK)
            def _():
                base = cid * CK
                pltpu.sync_copy(idx_hbm.at[pl.ds(base, CK)], idx_v)
                pltpu.sync_copy(data.at[pl.ds(base, CK)], buf)
                pltpu.sync_copy(buf, acc.at[idx_v], add=True)

            return carry

        lax.fori_loop(0, ITERS, body, 0)
        plsc.subcore_barrier()
        pltpu.sync_copy(acc.at[pl.ds(ib, RPT8)], out.at[c, pl.ds(ib, RPT8)])

        @pl.when(s == NS - 1)
        def _drain_tail():
            pltpu.sync_copy(acc.at[pl.ds(NS * RPT8, NTAIL)],
                            out.at[c, pl.ds(NS * RPT8, NTAIL)])

    return k


def _gather2(ta, tb, ia, ib):
    return _make_gather2()(ta, tb, ia, ib)


def _scatter(data, idx, zeros):
    return _make_scatter()(data, idx, zeros)


def _full_spec(shape):
    nd = len(shape)
    return pl.BlockSpec(shape, lambda i, _n=nd: (0,) * _n)


def _center_body(x15_ref, m1_ref, m2_ref, xc15_ref, cent15_ref):
    x15 = x15_ref[...]
    cent = jnp.dot(jnp.dot(x15, m1_ref[...], preferred_element_type=jnp.float32),
                   m2_ref[...], preferred_element_type=jnp.float32)
    cent15_ref[...] = cent
    xc15_ref[...] = x15 - cent


def _center(x15, m1, m2):
    g = x15.shape[0]
    return pl.pallas_call(
        _center_body,
        grid=(1,),
        in_specs=[_full_spec((g, 15)), _full_spec((15, 3)), _full_spec((3, 15))],
        out_specs=[_full_spec((g, 15)), _full_spec((g, 15))],
        out_shape=[jax.ShapeDtypeStruct((g, 15), jnp.float32),
                   jax.ShapeDtypeStruct((g, 15), jnp.float32)],
    )(x15, m1, m2)


def _embed_body(h0_ref, xc_ref, vel_ref, wemb_ref, bemb_ref, wr_ref, wc_ref,
                h_ref, rtab_ref, ctab_ref):
    h = jnp.dot(h0_ref[...], wemb_ref[...], preferred_element_type=jnp.float32) \
        + bemb_ref[...]
    h_ref[...] = h
    xc = xc_ref[...]
    vel = vel_ref[...]
    z = jnp.zeros((h.shape[0], DG - VO - 3), jnp.float32)
    rtab_ref[...] = jnp.concatenate(
        [jnp.dot(h, wr_ref[...], preferred_element_type=jnp.float32), xc, vel, z],
        axis=1)
    ctab_ref[...] = jnp.concatenate(
        [jnp.dot(h, wc_ref[...], preferred_element_type=jnp.float32), xc, vel, z],
        axis=1)


def _embed(h0, xc, vel, wemb, bemb, wr, wc):
    grid = N // BN
    return pl.pallas_call(
        _embed_body,
        grid=(grid,),
        in_specs=[
            pl.BlockSpec((BN, HID), lambda i: (i, 0)),
            pl.BlockSpec((BN, 3), lambda i: (i, 0)),
            pl.BlockSpec((BN, 3), lambda i: (i, 0)),
            _full_spec((HID, HID)),
            _full_spec((1, HID)),
            _full_spec((HID, HID)),
            _full_spec((HID, HID)),
        ],
        out_specs=[
            pl.BlockSpec((BN, HID), lambda i: (i, 0)),
            pl.BlockSpec((BN, DG), lambda i: (i, 0)),
            pl.BlockSpec((BN, DG), lambda i: (i, 0)),
        ],
        out_shape=[jax.ShapeDtypeStruct((N, HID), jnp.float32),
                   jax.ShapeDtypeStruct((N, DG), jnp.float32),
                   jax.ShapeDtypeStruct((N, DG), jnp.float32)],
    )(h0, xc, vel, wemb, bemb, wr, wc)


def _cols3(arr, off):
    return (arr[:, off:off + 1], arr[:, off + 1:off + 2], arr[:, off + 2:off + 3])


def _edge0_body(gr_ref, gc_ref, ea_ref, wf1_ref, bf1_ref, wf2_ref, bf2_ref,
                wef_ref, wrad_ref, b1_ref, w2_ref, b2_ref, wc1_ref, bc1_ref,
                wc2_ref, m_ref, t_ref, ef_ref):
    gr = gr_ref[...]
    gc = gc_ref[...]
    xr = _cols3(gr, XO)
    vr = _cols3(gr, VO)
    xl = _cols3(gc, XO)
    vc = _cols3(gc, VO)
    cd, cc, cv, radial = _frame_cols(xr, xl)

    def proj(v):
        return (cd[0] * v[0] + cd[1] * v[1] + cd[2] * v[2],
                cc[0] * v[0] + cc[1] * v[1] + cc[2] * v[2],
                cv[0] * v[0] + cv[1] * v[1] + cv[2] * v[2])

    ci = proj(xr)
    cj = proj(xl)
    vi = proj(vr)
    vj = proj(vc)
    ni = jnp.sqrt(ci[0] * ci[0] + ci[1] * ci[1] + ci[2] * ci[2])
    nj = jnp.sqrt(cj[0] * cj[0] + cj[1] * cj[1] + cj[2] * cj[2])
    cos = (ci[0] * cj[0] + ci[1] * cj[1] + ci[2] * cj[2]) / (ni + 1e-05) / (nj + 1e-05)
    sin = jnp.sqrt(jnp.clip(1.0 - cos * cos, 0.0, None))
    feat = jnp.concatenate(
        [ea_ref[...], sin, cos, ci[0], ci[1], ci[2], cj[0], cj[1], cj[2],
         vi[0], vi[1], vi[2], vj[0], vj[1], vj[2]], axis=1)
    e1 = _silu(jnp.dot(feat, wf1_ref[...], preferred_element_type=jnp.float32)
               + bf1_ref[...])
    ef = _silu(jnp.dot(e1, wf2_ref[...], preferred_element_type=jnp.float32)
               + bf2_ref[...])
    ef_ref[...] = ef

    hs = gr[:, :HID] + gc[:, :HID]
    z1 = hs + radial * wrad_ref[...] + b1_ref[...] \
        + jnp.dot(ef, wef_ref[...], preferred_element_type=jnp.float32)
    a1 = _silu(z1)
    m = _silu(jnp.dot(a1, w2_ref[...], preferred_element_type=jnp.float32)
              + b2_ref[...])
    c1 = _silu(jnp.dot(m, wc1_ref[...], preferred_element_type=jnp.float32)
               + bc1_ref[...])
    coff = jnp.dot(c1, wc2_ref[...], preferred_element_type=jnp.float32)
    c0, c1_, c2 = coff[:, 0:1], coff[:, 1:2], coff[:, 2:3]
    tx = jnp.clip(cd[0] * c0 + cc[0] * c1_ + cv[0] * c2, -100.0, 100.0)
    ty = jnp.clip(cd[1] * c0 + cc[1] * c1_ + cv[1] * c2, -100.0, 100.0)
    tz = jnp.clip(cd[2] * c0 + cc[2] * c1_ + cv[2] * c2, -100.0, 100.0)
    one = jnp.ones_like(tx)
    pad = jnp.zeros((gr.shape[0], DS - 4), jnp.float32)
    m_ref[...] = m
    t_ref[...] = jnp.concatenate([tx, ty, tz, one, pad], axis=1)


def _edge0_mlp(gr, gc, ea, wf1, bf1, wf2, bf2,
               wef, wrad, b1, w2, b2, wc1, bc1, wc2):
    grid = E // BE
    return pl.pallas_call(
        _edge0_body,
        grid=(grid,),
        in_specs=[
            pl.BlockSpec((BE, DG), lambda i: (i, 0)),
            pl.BlockSpec((BE, DG), lambda i: (i, 0)),
            pl.BlockSpec((BE, 2), lambda i: (i, 0)),
            _full_spec((16, HALF)),
            _full_spec((1, HALF)),
            _full_spec((HALF, HALF)),
            _full_spec((1, HALF)),
            _full_spec((HALF, HID)),
            _full_spec((1, HID)),
            _full_spec((1, HID)),
            _full_spec((HID, HID)),
            _full_spec((1, HID)),
            _full_spec((HID, HID)),
            _full_spec((1, HID)),
            _full_spec((HID, 3)),
        ],
        out_specs=[pl.BlockSpec((BE, HID), lambda i: (i, 0)),
                   pl.BlockSpec((BE, DS), lambda i: (i, 0)),
                   pl.BlockSpec((BE, HALF), lambda i: (i, 0))],
        out_shape=[jax.ShapeDtypeStruct((E, HID), jnp.float32),
                   jax.ShapeDtypeStruct((E, DS), jnp.float32),
                   jax.ShapeDtypeStruct((E, HALF), jnp.float32)],
    )(gr, gc, ea, wf1, bf1, wf2, bf2, wef, wrad, b1, w2, b2, wc1, bc1, wc2)


def _edge_body(last, gr_ref, gc_ref, ef_ref, wef_ref, wrad_ref, b1_ref,
               w2_ref, b2_ref, wc1_ref, bc1_ref, wc2_ref, *out_refs):
    gr = gr_ref[...]
    gc = gc_ref[...]
    hs = gr[:, :HID] + gc[:, :HID]
    xr = _cols3(gr, XO)
    xl = _cols3(gc, XO)
    cd, cc, cv, radial = _frame_cols(xr, xl)
    z1 = hs + radial * wrad_ref[...] + b1_ref[...] \
        + jnp.dot(ef_ref[...], wef_ref[...], preferred_element_type=jnp.float32)
    a1 = _silu(z1)
    m = _silu(jnp.dot(a1, w2_ref[...], preferred_element_type=jnp.float32)
              + b2_ref[...])
    c1 = _silu(jnp.dot(m, wc1_ref[...], preferred_element_type=jnp.float32)
               + bc1_ref[...])
    coff = jnp.dot(c1, wc2_ref[...], preferred_element_type=jnp.float32)
    c0, c1_, c2 = coff[:, 0:1], coff[:, 1:2], coff[:, 2:3]
    tx = jnp.clip(cd[0] * c0 + cc[0] * c1_ + cv[0] * c2, -100.0, 100.0)
    ty = jnp.clip(cd[1] * c0 + cc[1] * c1_ + cv[1] * c2, -100.0, 100.0)
    tz = jnp.clip(cd[2] * c0 + cc[2] * c1_ + cv[2] * c2, -100.0, 100.0)
    one = jnp.ones_like(tx)
    pad = jnp.zeros((gr.shape[0], DS - 4), jnp.float32)
    tvec = jnp.concatenate([tx, ty, tz, one, pad], axis=1)
    if last:
        out_refs[0][...] = tvec
    else:
        out_refs[0][...] = m
        out_refs[1][...] = tvec


def _edge_mlp(last, gr, gc, ef, wef, wrad, b1, w2, b2, wc1, bc1, wc2):
    grid = E // BE
    if last:
        out_specs = [pl.BlockSpec((BE, DS), lambda i: (i, 0))]
        out_shape = [jax.ShapeDtypeStruct((E, DS), jnp.float32)]
    else:
        out_specs = [pl.BlockSpec((BE, HID), lambda i: (i, 0)),
                     pl.BlockSpec((BE, DS), lambda i: (i, 0))]
        out_shape = [jax.ShapeDtypeStruct((E, HID), jnp.float32),
                     jax.ShapeDtypeStruct((E, DS), jnp.float32)]
    return pl.pallas_call(
        functools.partial(_edge_body, last),
        grid=(grid,),
        in_specs=[
            pl.BlockSpec((BE, DG), lambda i: (i, 0)),
            pl.BlockSpec((BE, DG), lambda i: (i, 0)),
            pl.BlockSpec((BE, HALF), lambda i: (i, 0)),
            _full_spec((HALF, HID)),
            _full_spec((1, HID)),
            _full_spec((1, HID)),
            _full_spec((HID, HID)),
            _full_spec((1, HID)),
            _full_spec((HID, HID)),
            _full_spec((1, HID)),
            _full_spec((HID, 3)),
        ],
        out_specs=out_specs,
        out_shape=out_shape,
    )(gr, gc, ef, wef, wrad, b1, w2, b2, wc1, bc1, wc2)


def _node_mid_body(h_ref, mpm_ref, mpt_ref, xc_ref, vel_ref,
                   wv1_ref, bv1_ref, wv2_ref, bv2_ref,
                   wn1h_ref, wn1g_ref, bn1_ref, wn2_ref, bn2_ref,
                   wrn_ref, wcn_ref,
                   hn_ref, xcn_ref, rtab_ref, ctab_ref):
    h = h_ref[...]
    mpm = mpm_ref[...]
    mpt = mpt_ref[...]
    hag = mpm[0] + mpm[1]
    tsum = mpt[0] + mpt[1]
    agg = tsum[:, 0:3]
    cnt = tsum[:, 3:4]
    xc = xc_ref[...] + agg / jnp.maximum(cnt, 1.0)
    vmul = jnp.dot(_silu(jnp.dot(h, wv1_ref[...], preferred_element_type=jnp.float32)
                         + bv1_ref[...]),
                   wv2_ref[...], preferred_element_type=jnp.float32) + bv2_ref[...]
    xc = xc + vmul * vel_ref[...]
    t = _silu(jnp.dot(h, wn1h_ref[...], preferred_element_type=jnp.float32)
              + jnp.dot(hag, wn1g_ref[...], preferred_element_type=jnp.float32)
              + bn1_ref[...])
    hn = h + jnp.dot(t, wn2_ref[...], preferred_element_type=jnp.float32) + bn2_ref[...]
    hn_ref[...] = hn
    xcn_ref[...] = xc
    z = jnp.zeros((h.shape[0], DG - XO - 3), jnp.float32)
    rtab_ref[...] = jnp.concatenate(
        [jnp.dot(hn, wrn_ref[...], preferred_element_type=jnp.float32), xc, z], axis=1)
    ctab_ref[...] = jnp.concatenate(
        [jnp.dot(hn, wcn_ref[...], preferred_element_type=jnp.float32), xc, z], axis=1)


def _node_mid(h, mpm, mpt, xc, vel, wv1, bv1, wv2, bv2,
              wn1h, wn1g, bn1, wn2, bn2, wrn, wcn):
    grid = N // BN
    return pl.pallas_call(
        _node_mid_body,
        grid=(grid,),
        in_specs=[
            pl.BlockSpec((BN, HID), lambda i: (i, 0)),
            pl.BlockSpec((NC, BN, HID), lambda i: (0, i, 0)),
            pl.BlockSpec((NC, BN, DS), lambda i: (0, i, 0)),
            pl.BlockSpec((BN, 3), lambda i: (i, 0)),
            pl.BlockSpec((BN, 3), lambda i: (i, 0)),
            _full_spec((HID, HID)),
            _full_spec((1, HID)),
            _full_spec((HID, 1)),
            _full_spec((1, 1)),
            _full_spec((HID, HID)),
            _full_spec((HID, HID)),
            _full_spec((1, HID)),
            _full_spec((HID, HID)),
            _full_spec((1, HID)),
            _full_spec((HID, HID)),
            _full_spec((HID, HID)),
        ],
        out_specs=[
            pl.BlockSpec((BN, HID), lambda i: (i, 0)),
            pl.BlockSpec((BN, 3), lambda i: (i, 0)),
            pl.BlockSpec((BN, DG), lambda i: (i, 0)),
            pl.BlockSpec((BN, DG), lambda i: (i, 0)),
        ],
        out_shape=[jax.ShapeDtypeStruct((N, HID), jnp.float32),
                   jax.ShapeDtypeStruct((N, 3), jnp.float32),
                   jax.ShapeDtypeStruct((N, DG), jnp.float32),
                   jax.ShapeDtypeStruct((N, DG), jnp.float32)],
    )(h, mpm, mpt, xc, vel, wv1, bv1, wv2, bv2, wn1h, wn1g, bn1, wn2, bn2, wrn, wcn)


def _node_last_body(h_ref, mpt_ref, xc_ref, vel_ref, cent_ref,
                    wv1_ref, bv1_ref, wv2_ref, bv2_ref, out_ref):
    h = h_ref[...]
    mpt = mpt_ref[...]
    tsum = mpt[0] + mpt[1]
    agg = tsum[:, 0:3]
    cnt = tsum[:, 3:4]
    xc = xc_ref[...] + agg / jnp.maximum(cnt, 1.0)
    vmul = jnp.dot(_silu(jnp.dot(h, wv1_ref[...], preferred_element_type=jnp.float32)
                         + bv1_ref[...]),
                   wv2_ref[...], preferred_element_type=jnp.float32) + bv2_ref[...]
    xc = xc + vmul * vel_ref[...]
    out_ref[...] = xc + cent_ref[...]


def _node_last(h, mpt, xc, vel, cent, wv1, bv1, wv2, bv2):
    grid = N // BN
    return pl.pallas_call(
        _node_last_body,
        grid=(grid,),
        in_specs=[
            pl.BlockSpec((BN, HID), lambda i: (i, 0)),
            pl.BlockSpec((NC, BN, DS), lambda i: (0, i, 0)),
            pl.BlockSpec((BN, 3), lambda i: (i, 0)),
            pl.BlockSpec((BN, 3), lambda i: (i, 0)),
            pl.BlockSpec((BN, 3), lambda i: (i, 0)),
            _full_spec((HID, HID)),
            _full_spec((1, HID)),
            _full_spec((HID, 1)),
            _full_spec((1, 1)),
        ],
        out_specs=pl.BlockSpec((BN, 3), lambda i: (i, 0)),
        out_shape=jax.ShapeDtypeStruct((N, 3), jnp.float32),
    )(h, mpt, xc, vel, cent, wv1, bv1, wv2, bv2)


def kernel(h, x, vel, edge_attr, params, edges):
    row = edges[0]
    col = edges[1]
    f32 = jnp.float32

    m1 = jnp.zeros((15, 3), f32).at[jnp.arange(15), jnp.arange(15) % 3].set(0.2)
    m2 = jnp.zeros((3, 15), f32).at[jnp.arange(15) % 3, jnp.arange(15)].set(1.0)

    x15 = x.reshape(N // N_POINTS, 15)
    xc15, cent15 = _center(x15, m1, m2)
    xc = xc15.reshape(N, 3)
    cent = cent15.reshape(N, 3)

    we1 = params['We1']
    wr0 = we1[0, 0:HID, :]
    wc0 = we1[0, HID:2 * HID, :]
    hh, rtab, ctab = _embed(
        h, xc, vel, params['Wemb'], params['bemb'].reshape(1, HID), wr0, wc0)

    zeros_s = jnp.zeros((N, DS), f32)
    ef = None

    for l in range(N_LAYERS):
        last = l == N_LAYERS - 1
        gr, gc = _gather2(rtab, ctab, row, col)
        if l == 0:
            m0, t0, ef = _edge0_mlp(
                gr, gc, edge_attr,
                params['Wf1'], params['bf1'].reshape(1, HALF),
                params['Wf2'], params['bf2'].reshape(1, HALF),
                we1[l, 2 * HID + 1:, :], we1[l, 2 * HID:2 * HID + 1, :],
                params['be1'][l].reshape(1, HID),
                params['We2'][l], params['be2'][l].reshape(1, HID),
                params['Wc1'][l], params['bc1'][l].reshape(1, HID),
                params['Wc2'][l])
            ed = (m0, t0)
        else:
            ed = _edge_mlp(
                last, gr, gc, ef,
                we1[l, 2 * HID + 1:, :], we1[l, 2 * HID:2 * HID + 1, :],
                params['be1'][l].reshape(1, HID),
                params['We2'][l], params['be2'][l].reshape(1, HID),
                params['Wc1'][l], params['bc1'][l].reshape(1, HID),
                params['Wc2'][l])
        if last:
            mpt = _scatter(ed[0], row, zeros_s)
            out = _node_last(
                hh, mpt, xc, vel, cent,
                params['Wv1'][l], params['bv1'][l].reshape(1, HID),
                params['Wv2'][l], params['bv2'][l].reshape(1, 1))
        else:
            mpm = _scatter(ed[0], row, zeros_s)
            mpt = _scatter(ed[1], row, zeros_s)
            hh, xc, rtab, ctab = _node_mid(
                hh, mpm, mpt, xc, vel,
                params['Wv1'][l], params['bv1'][l].reshape(1, HID),
                params['Wv2'][l], params['bv2'][l].reshape(1, 1),
                params['Wn1'][l][0:HID, :], params['Wn1'][l][HID:, :],
                params['bn1'][l].reshape(1, HID),
                params['Wn2'][l], params['bn2'][l].reshape(1, HID),
                we1[l + 1, 0:HID, :], we1[l + 1, HID:2 * HID, :])
    return out


# speedup vs baseline: 2.2328x; 1.1543x over previous
import functools

import jax
import jax.numpy as jnp
from jax import lax
from jax.experimental import pallas as pl
from jax.experimental.pallas import tpu as pltpu
from jax.experimental.pallas import tpu_sc as plsc

N = 10000
E = 320000
HID = 128
HALF = 64
N_LAYERS = 4
N_POINTS = 5

NC = 2
NS = 16
NW = NC * NS
CK = 128
EH = E // 2
RPT8 = 624
NTAIL = N - NS * RPT8

DG = 256
DS = 128
XO = 128
VO = 131

BE = 1280
BN = 2000


def _sc_mesh():
    return plsc.VectorSubcoreMesh(core_axis_name="c", subcore_axis_name="s",
                                  num_cores=NC, num_subcores=NS)


def _silu(v):
    return v * (1.0 / (1.0 + jnp.exp(-v)))


def _cross(ax, ay, az, bx, by, bz):
    return (ay * bz - az * by, az * bx - ax * bz, ax * by - ay * bx)


def _frame_cols(xr, xl):
    dx, dy, dz = xr[0] - xl[0], xr[1] - xl[1], xr[2] - xl[2]
    radial = dx * dx + dy * dy + dz * dz
    nrm = jnp.sqrt(radial) + 1.0
    cd = (dx / nrm, dy / nrm, dz / nrm)
    cx, cy, cz = _cross(xr[0], xr[1], xr[2], xl[0], xl[1], xl[2])
    cn = jnp.sqrt(cx * cx + cy * cy + cz * cz) + 1.0
    cc = (cx / cn, cy / cn, cz / cn)
    cv = _cross(cd[0], cd[1], cd[2], cc[0], cc[1], cc[2])
    return cd, cc, cv, radial


@functools.lru_cache(maxsize=None)
def _make_gather2(ne):
    nchunk = ne // CK
    iters = -(-nchunk // NW)

    @functools.partial(
        pl.kernel,
        out_type=(jax.ShapeDtypeStruct((ne, DG), jnp.float32),
                  jax.ShapeDtypeStruct((ne, DG), jnp.float32)),
        mesh=_sc_mesh(),
        scratch_types=[
            pltpu.VMEM((CK,), jnp.int32),
            pltpu.VMEM((CK,), jnp.int32),
            pltpu.VMEM((CK, DG), jnp.float32),
            pltpu.VMEM((CK, DG), jnp.float32),
            pltpu.SemaphoreType.DMA,
            pltpu.SemaphoreType.DMA,
        ],
    )
    def k(ta, tb, ia, ib, oa, ob, idx_a, idx_b, buf_a, buf_b, sem_a, sem_b):
        wid = lax.axis_index("s") * NC + lax.axis_index("c")

        def body(i, carry):
            cid = wid + i * NW

            @pl.when(cid < nchunk)
            def _():
                base = cid * CK
                pltpu.sync_copy(ia.at[pl.ds(base, CK)], idx_a)
                pltpu.sync_copy(ib.at[pl.ds(base, CK)], idx_b)
                ca = pltpu.async_copy(ta.at[idx_a], buf_a, sem_a)
                cb = pltpu.async_copy(tb.at[idx_b], buf_b, sem_b)
                ca.wait()
                cb.wait()
                pltpu.sync_copy(buf_a, oa.at[pl.ds(base, CK)])
                pltpu.sync_copy(buf_b, ob.at[pl.ds(base, CK)])

            return carry

        lax.fori_loop(0, iters, body, 0)

    return k


@functools.lru_cache(maxsize=None)
def _make_scatter(ne):
    nchunk = ne // CK
    iters = -(-nchunk // NW)

    @functools.partial(
        pl.kernel,
        out_type=jax.ShapeDtypeStruct((NC, N, DS), jnp.float32),
        mesh=_sc_mesh(),
        scratch_types=[
            pltpu.VMEM((CK,), jnp.int32),
            pltpu.VMEM((CK, DS), jnp.float32),
            pltpu.VMEM_SHARED((N, DS), jnp.float32),
        ],
    )
    def k(data, idx_hbm, zeros, out, idx_v, buf, acc):
        c = lax.axis_index("c")
        s = lax.axis_index("s")
        wid = s * NC + c
        ib = s * RPT8
        pltpu.sync_copy(zeros.at[pl.ds(ib, RPT8)], acc.at[pl.ds(ib, RPT8)])

        @pl.when(s == NS - 1)
        def _init_tail():
            pltpu.sync_copy(zeros.at[pl.ds(NS * RPT8, NTAIL)],
                            acc.at[pl.ds(NS * RPT8, NTAIL)])

        plsc.subcore_barrier()

        def body(i, carry):
            cid = wid + i * NW

            @pl.when(cid < nchunk)
            def _():
                base = cid * CK
                pltpu.sync_copy(idx_hbm.at[pl.ds(base, CK)], idx_v)
                pltpu.sync_copy(data.at[pl.ds(base, CK)], buf)
                pltpu.sync_copy(buf, acc.at[idx_v], add=True)

            return carry

        lax.fori_loop(0, iters, body, 0)
        plsc.subcore_barrier()
        pltpu.sync_copy(acc.at[pl.ds(ib, RPT8)], out.at[c, pl.ds(ib, RPT8)])

        @pl.when(s == NS - 1)
        def _drain_tail():
            pltpu.sync_copy(acc.at[pl.ds(NS * RPT8, NTAIL)],
                            out.at[c, pl.ds(NS * RPT8, NTAIL)])

    return k


def _gather2(ta, tb, ia, ib):
    return _make_gather2(ia.shape[0])(ta, tb, ia, ib)


def _scatter(data, idx, zeros):
    return _make_scatter(idx.shape[0])(data, idx, zeros)


def _full_spec(shape):
    nd = len(shape)
    return pl.BlockSpec(shape, lambda i, _n=nd: (0,) * _n)


def _center_body(x15_ref, m1_ref, m2_ref, xc15_ref, cent15_ref):
    x15 = x15_ref[...]
    cent = jnp.dot(jnp.dot(x15, m1_ref[...], preferred_element_type=jnp.float32),
                   m2_ref[...], preferred_element_type=jnp.float32)
    cent15_ref[...] = cent
    xc15_ref[...] = x15 - cent


def _center(x15, m1, m2):
    g = x15.shape[0]
    return pl.pallas_call(
        _center_body,
        grid=(1,),
        in_specs=[_full_spec((g, 15)), _full_spec((15, 3)), _full_spec((3, 15))],
        out_specs=[_full_spec((g, 15)), _full_spec((g, 15))],
        out_shape=[jax.ShapeDtypeStruct((g, 15), jnp.float32),
                   jax.ShapeDtypeStruct((g, 15), jnp.float32)],
    )(x15, m1, m2)


def _embed_body(h0_ref, xc_ref, vel_ref, wemb_ref, bemb_ref, wr_ref, wc_ref,
                h_ref, rtab_ref, ctab_ref):
    h = jnp.dot(h0_ref[...], wemb_ref[...], preferred_element_type=jnp.float32) \
        + bemb_ref[...]
    h_ref[...] = h
    xc = xc_ref[...]
    vel = vel_ref[...]
    z = jnp.zeros((h.shape[0], DG - VO - 3), jnp.float32)
    rtab_ref[...] = jnp.concatenate(
        [jnp.dot(h, wr_ref[...], preferred_element_type=jnp.float32), xc, vel, z],
        axis=1)
    ctab_ref[...] = jnp.concatenate(
        [jnp.dot(h, wc_ref[...], preferred_element_type=jnp.float32), xc, vel, z],
        axis=1)


def _embed(h0, xc, vel, wemb, bemb, wr, wc):
    grid = N // BN
    return pl.pallas_call(
        _embed_body,
        grid=(grid,),
        in_specs=[
            pl.BlockSpec((BN, HID), lambda i: (i, 0)),
            pl.BlockSpec((BN, 3), lambda i: (i, 0)),
            pl.BlockSpec((BN, 3), lambda i: (i, 0)),
            _full_spec((HID, HID)),
            _full_spec((1, HID)),
            _full_spec((HID, HID)),
            _full_spec((HID, HID)),
        ],
        out_specs=[
            pl.BlockSpec((BN, HID), lambda i: (i, 0)),
            pl.BlockSpec((BN, DG), lambda i: (i, 0)),
            pl.BlockSpec((BN, DG), lambda i: (i, 0)),
        ],
        out_shape=[jax.ShapeDtypeStruct((N, HID), jnp.float32),
                   jax.ShapeDtypeStruct((N, DG), jnp.float32),
                   jax.ShapeDtypeStruct((N, DG), jnp.float32)],
    )(h0, xc, vel, wemb, bemb, wr, wc)


def _cols3(arr, off):
    return (arr[:, off:off + 1], arr[:, off + 1:off + 2], arr[:, off + 2:off + 3])


def _edge0_body(gr_ref, gc_ref, ea_ref, wf1_ref, bf1_ref, wf2_ref, bf2_ref,
                wef_ref, wrad_ref, b1_ref, w2_ref, b2_ref, wc1_ref, bc1_ref,
                wc2_ref, m_ref, t_ref, ef_ref):
    gr = gr_ref[...]
    gc = gc_ref[...]
    xr = _cols3(gr, XO)
    vr = _cols3(gr, VO)
    xl = _cols3(gc, XO)
    vc = _cols3(gc, VO)
    cd, cc, cv, radial = _frame_cols(xr, xl)

    def proj(v):
        return (cd[0] * v[0] + cd[1] * v[1] + cd[2] * v[2],
                cc[0] * v[0] + cc[1] * v[1] + cc[2] * v[2],
                cv[0] * v[0] + cv[1] * v[1] + cv[2] * v[2])

    ci = proj(xr)
    cj = proj(xl)
    vi = proj(vr)
    vj = proj(vc)
    ni = jnp.sqrt(ci[0] * ci[0] + ci[1] * ci[1] + ci[2] * ci[2])
    nj = jnp.sqrt(cj[0] * cj[0] + cj[1] * cj[1] + cj[2] * cj[2])
    cos = (ci[0] * cj[0] + ci[1] * cj[1] + ci[2] * cj[2]) / (ni + 1e-05) / (nj + 1e-05)
    sin = jnp.sqrt(jnp.clip(1.0 - cos * cos, 0.0, None))
    feat = jnp.concatenate(
        [ea_ref[...], sin, cos, ci[0], ci[1], ci[2], cj[0], cj[1], cj[2],
         vi[0], vi[1], vi[2], vj[0], vj[1], vj[2]], axis=1)
    e1 = _silu(jnp.dot(feat, wf1_ref[...], preferred_element_type=jnp.float32)
               + bf1_ref[...])
    ef = _silu(jnp.dot(e1, wf2_ref[...], preferred_element_type=jnp.float32)
               + bf2_ref[...])
    ef_ref[...] = ef

    hs = gr[:, :HID] + gc[:, :HID]
    z1 = hs + radial * wrad_ref[...] + b1_ref[...] \
        + jnp.dot(ef, wef_ref[...], preferred_element_type=jnp.float32)
    a1 = _silu(z1)
    m = _silu(jnp.dot(a1, w2_ref[...], preferred_element_type=jnp.float32)
              + b2_ref[...])
    c1 = _silu(jnp.dot(m, wc1_ref[...], preferred_element_type=jnp.float32)
               + bc1_ref[...])
    coff = jnp.dot(c1, wc2_ref[...], preferred_element_type=jnp.float32)
    c0, c1_, c2 = coff[:, 0:1], coff[:, 1:2], coff[:, 2:3]
    tx = jnp.clip(cd[0] * c0 + cc[0] * c1_ + cv[0] * c2, -100.0, 100.0)
    ty = jnp.clip(cd[1] * c0 + cc[1] * c1_ + cv[1] * c2, -100.0, 100.0)
    tz = jnp.clip(cd[2] * c0 + cc[2] * c1_ + cv[2] * c2, -100.0, 100.0)
    one = jnp.ones_like(tx)
    pad = jnp.zeros((gr.shape[0], DS - 4), jnp.float32)
    m_ref[...] = m
    t_ref[...] = jnp.concatenate([tx, ty, tz, one, pad], axis=1)


def _edge0_mlp(gr, gc, ea, wf1, bf1, wf2, bf2,
               wef, wrad, b1, w2, b2, wc1, bc1, wc2):
    ne = gr.shape[0]
    grid = ne // BE
    return pl.pallas_call(
        _edge0_body,
        grid=(grid,),
        in_specs=[
            pl.BlockSpec((BE, DG), lambda i: (i, 0)),
            pl.BlockSpec((BE, DG), lambda i: (i, 0)),
            pl.BlockSpec((BE, 2), lambda i: (i, 0)),
            _full_spec((16, HALF)),
            _full_spec((1, HALF)),
            _full_spec((HALF, HALF)),
            _full_spec((1, HALF)),
            _full_spec((HALF, HID)),
            _full_spec((1, HID)),
            _full_spec((1, HID)),
            _full_spec((HID, HID)),
            _full_spec((1, HID)),
            _full_spec((HID, HID)),
            _full_spec((1, HID)),
            _full_spec((HID, 3)),
        ],
        out_specs=[pl.BlockSpec((BE, HID), lambda i: (i, 0)),
                   pl.BlockSpec((BE, DS), lambda i: (i, 0)),
                   pl.BlockSpec((BE, HALF), lambda i: (i, 0))],
        out_shape=[jax.ShapeDtypeStruct((ne, HID), jnp.float32),
                   jax.ShapeDtypeStruct((ne, DS), jnp.float32),
                   jax.ShapeDtypeStruct((ne, HALF), jnp.float32)],
    )(gr, gc, ea, wf1, bf1, wf2, bf2, wef, wrad, b1, w2, b2, wc1, bc1, wc2)


def _edge_body(last, gr_ref, gc_ref, ef_ref, wef_ref, wrad_ref, b1_ref,
               w2_ref, b2_ref, wc1_ref, bc1_ref, wc2_ref, *out_refs):
    gr = gr_ref[...]
    gc = gc_ref[...]
    hs = gr[:, :HID] + gc[:, :HID]
    xr = _cols3(gr, XO)
    xl = _cols3(gc, XO)
    cd, cc, cv, radial = _frame_cols(xr, xl)
    z1 = hs + radial * wrad_ref[...] + b1_ref[...] \
        + jnp.dot(ef_ref[...], wef_ref[...], preferred_element_type=jnp.float32)
    a1 = _silu(z1)
    m = _silu(jnp.dot(a1, w2_ref[...], preferred_element_type=jnp.float32)
              + b2_ref[...])
    c1 = _silu(jnp.dot(m, wc1_ref[...], preferred_element_type=jnp.float32)
               + bc1_ref[...])
    coff = jnp.dot(c1, wc2_ref[...], preferred_element_type=jnp.float32)
    c0, c1_, c2 = coff[:, 0:1], coff[:, 1:2], coff[:, 2:3]
    tx = jnp.clip(cd[0] * c0 + cc[0] * c1_ + cv[0] * c2, -100.0, 100.0)
    ty = jnp.clip(cd[1] * c0 + cc[1] * c1_ + cv[1] * c2, -100.0, 100.0)
    tz = jnp.clip(cd[2] * c0 + cc[2] * c1_ + cv[2] * c2, -100.0, 100.0)
    one = jnp.ones_like(tx)
    pad = jnp.zeros((gr.shape[0], DS - 4), jnp.float32)
    tvec = jnp.concatenate([tx, ty, tz, one, pad], axis=1)
    if last:
        out_refs[0][...] = tvec
    else:
        out_refs[0][...] = m
        out_refs[1][...] = tvec


def _edge_mlp(last, gr, gc, ef, wef, wrad, b1, w2, b2, wc1, bc1, wc2):
    ne = gr.shape[0]
    grid = ne // BE
    if last:
        out_specs = [pl.BlockSpec((BE, DS), lambda i: (i, 0))]
        out_shape = [jax.ShapeDtypeStruct((ne, DS), jnp.float32)]
    else:
        out_specs = [pl.BlockSpec((BE, HID), lambda i: (i, 0)),
                     pl.BlockSpec((BE, DS), lambda i: (i, 0))]
        out_shape = [jax.ShapeDtypeStruct((ne, HID), jnp.float32),
                     jax.ShapeDtypeStruct((ne, DS), jnp.float32)]
    return pl.pallas_call(
        functools.partial(_edge_body, last),
        grid=(grid,),
        in_specs=[
            pl.BlockSpec((BE, DG), lambda i: (i, 0)),
            pl.BlockSpec((BE, DG), lambda i: (i, 0)),
            pl.BlockSpec((BE, HALF), lambda i: (i, 0)),
            _full_spec((HALF, HID)),
            _full_spec((1, HID)),
            _full_spec((1, HID)),
            _full_spec((HID, HID)),
            _full_spec((1, HID)),
            _full_spec((HID, HID)),
            _full_spec((1, HID)),
            _full_spec((HID, 3)),
        ],
        out_specs=out_specs,
        out_shape=out_shape,
    )(gr, gc, ef, wef, wrad, b1, w2, b2, wc1, bc1, wc2)


def _node_mid_body(h_ref, mpm0_ref, mpm1_ref, mpt0_ref, mpt1_ref,
                   xc_ref, vel_ref,
                   wv1_ref, bv1_ref, wv2_ref, bv2_ref,
                   wn1h_ref, wn1g_ref, bn1_ref, wn2_ref, bn2_ref,
                   wrn_ref, wcn_ref,
                   hn_ref, xcn_ref, rtab_ref, ctab_ref):
    h = h_ref[...]
    mpm0 = mpm0_ref[...]
    mpm1 = mpm1_ref[...]
    mpt0 = mpt0_ref[...]
    mpt1 = mpt1_ref[...]
    hag = mpm0[0] + mpm0[1] + mpm1[0] + mpm1[1]
    tsum = mpt0[0] + mpt0[1] + mpt1[0] + mpt1[1]
    agg = tsum[:, 0:3]
    cnt = tsum[:, 3:4]
    xc = xc_ref[...] + agg / jnp.maximum(cnt, 1.0)
    vmul = jnp.dot(_silu(jnp.dot(h, wv1_ref[...], preferred_element_type=jnp.float32)
                         + bv1_ref[...]),
                   wv2_ref[...], preferred_element_type=jnp.float32) + bv2_ref[...]
    xc = xc + vmul * vel_ref[...]
    t = _silu(jnp.dot(h, wn1h_ref[...], preferred_element_type=jnp.float32)
              + jnp.dot(hag, wn1g_ref[...], preferred_element_type=jnp.float32)
              + bn1_ref[...])
    hn = h + jnp.dot(t, wn2_ref[...], preferred_element_type=jnp.float32) + bn2_ref[...]
    hn_ref[...] = hn
    xcn_ref[...] = xc
    z = jnp.zeros((h.shape[0], DG - XO - 3), jnp.float32)
    rtab_ref[...] = jnp.concatenate(
        [jnp.dot(hn, wrn_ref[...], preferred_element_type=jnp.float32), xc, z], axis=1)
    ctab_ref[...] = jnp.concatenate(
        [jnp.dot(hn, wcn_ref[...], preferred_element_type=jnp.float32), xc, z], axis=1)


def _node_mid(h, mpm0, mpm1, mpt0, mpt1, xc, vel, wv1, bv1, wv2, bv2,
              wn1h, wn1g, bn1, wn2, bn2, wrn, wcn):
    grid = N // BN
    return pl.pallas_call(
        _node_mid_body,
        grid=(grid,),
        in_specs=[
            pl.BlockSpec((BN, HID), lambda i: (i, 0)),
            pl.BlockSpec((NC, BN, HID), lambda i: (0, i, 0)),
            pl.BlockSpec((NC, BN, HID), lambda i: (0, i, 0)),
            pl.BlockSpec((NC, BN, DS), lambda i: (0, i, 0)),
            pl.BlockSpec((NC, BN, DS), lambda i: (0, i, 0)),
            pl.BlockSpec((BN, 3), lambda i: (i, 0)),
            pl.BlockSpec((BN, 3), lambda i: (i, 0)),
            _full_spec((HID, HID)),
            _full_spec((1, HID)),
            _full_spec((HID, 1)),
            _full_spec((1, 1)),
            _full_spec((HID, HID)),
            _full_spec((HID, HID)),
            _full_spec((1, HID)),
            _full_spec((HID, HID)),
            _full_spec((1, HID)),
            _full_spec((HID, HID)),
            _full_spec((HID, HID)),
        ],
        out_specs=[
            pl.BlockSpec((BN, HID), lambda i: (i, 0)),
            pl.BlockSpec((BN, 3), lambda i: (i, 0)),
            pl.BlockSpec((BN, DG), lambda i: (i, 0)),
            pl.BlockSpec((BN, DG), lambda i: (i, 0)),
        ],
        out_shape=[jax.ShapeDtypeStruct((N, HID), jnp.float32),
                   jax.ShapeDtypeStruct((N, 3), jnp.float32),
                   jax.ShapeDtypeStruct((N, DG), jnp.float32),
                   jax.ShapeDtypeStruct((N, DG), jnp.float32)],
    )(h, mpm0, mpm1, mpt0, mpt1, xc, vel, wv1, bv1, wv2, bv2,
      wn1h, wn1g, bn1, wn2, bn2, wrn, wcn)


def _node_last_body(h_ref, mpt0_ref, mpt1_ref, xc_ref, vel_ref, cent_ref,
                    wv1_ref, bv1_ref, wv2_ref, bv2_ref, out_ref):
    h = h_ref[...]
    mpt0 = mpt0_ref[...]
    mpt1 = mpt1_ref[...]
    tsum = mpt0[0] + mpt0[1] + mpt1[0] + mpt1[1]
    agg = tsum[:, 0:3]
    cnt = tsum[:, 3:4]
    xc = xc_ref[...] + agg / jnp.maximum(cnt, 1.0)
    vmul = jnp.dot(_silu(jnp.dot(h, wv1_ref[...], preferred_element_type=jnp.float32)
                         + bv1_ref[...]),
                   wv2_ref[...], preferred_element_type=jnp.float32) + bv2_ref[...]
    xc = xc + vmul * vel_ref[...]
    out_ref[...] = xc + cent_ref[...]


def _node_last(h, mpt0, mpt1, xc, vel, cent, wv1, bv1, wv2, bv2):
    grid = N // BN
    return pl.pallas_call(
        _node_last_body,
        grid=(grid,),
        in_specs=[
            pl.BlockSpec((BN, HID), lambda i: (i, 0)),
            pl.BlockSpec((NC, BN, DS), lambda i: (0, i, 0)),
            pl.BlockSpec((NC, BN, DS), lambda i: (0, i, 0)),
            pl.BlockSpec((BN, 3), lambda i: (i, 0)),
            pl.BlockSpec((BN, 3), lambda i: (i, 0)),
            pl.BlockSpec((BN, 3), lambda i: (i, 0)),
            _full_spec((HID, HID)),
            _full_spec((1, HID)),
            _full_spec((HID, 1)),
            _full_spec((1, 1)),
        ],
        out_specs=pl.BlockSpec((BN, 3), lambda i: (i, 0)),
        out_shape=jax.ShapeDtypeStruct((N, 3), jnp.float32),
    )(h, mpt0, mpt1, xc, vel, cent, wv1, bv1, wv2, bv2)


def kernel(h, x, vel, edge_attr, params, edges):
    row = edges[0]
    col = edges[1]
    f32 = jnp.float32

    m1 = jnp.zeros((15, 3), f32).at[jnp.arange(15), jnp.arange(15) % 3].set(0.2)
    m2 = jnp.zeros((3, 15), f32).at[jnp.arange(15) % 3, jnp.arange(15)].set(1.0)

    x15 = x.reshape(N // N_POINTS, 15)
    xc15, cent15 = _center(x15, m1, m2)
    xc = xc15.reshape(N, 3)
    cent = cent15.reshape(N, 3)

    we1 = params['We1']
    wr0 = we1[0, 0:HID, :]
    wc0 = we1[0, HID:2 * HID, :]
    hh, rtab, ctab = _embed(
        h, xc, vel, params['Wemb'], params['bemb'].reshape(1, HID), wr0, wc0)

    zeros_s = jnp.zeros((N, DS), f32)
    rows = (row[:EH], row[EH:])
    cols = (col[:EH], col[EH:])
    eas = (edge_attr[:EH], edge_attr[EH:])
    efs = [None, None]

    for l in range(N_LAYERS):
        last = l == N_LAYERS - 1
        eds = []
        for hf in range(2):
            gr, gc = _gather2(rtab, ctab, rows[hf], cols[hf])
            if l == 0:
                m0, t0, efs[hf] = _edge0_mlp(
                    gr, gc, eas[hf],
                    params['Wf1'], params['bf1'].reshape(1, HALF),
                    params['Wf2'], params['bf2'].reshape(1, HALF),
                    we1[l, 2 * HID + 1:, :], we1[l, 2 * HID:2 * HID + 1, :],
                    params['be1'][l].reshape(1, HID),
                    params['We2'][l], params['be2'][l].reshape(1, HID),
                    params['Wc1'][l], params['bc1'][l].reshape(1, HID),
                    params['Wc2'][l])
                eds.append((m0, t0))
            else:
                eds.append(_edge_mlp(
                    last, gr, gc, efs[hf],
                    we1[l, 2 * HID + 1:, :], we1[l, 2 * HID:2 * HID + 1, :],
                    params['be1'][l].reshape(1, HID),
                    params['We2'][l], params['be2'][l].reshape(1, HID),
                    params['Wc1'][l], params['bc1'][l].reshape(1, HID),
                    params['Wc2'][l]))
        if last:
            mpt0 = _scatter(eds[0][0], rows[0], zeros_s)
            mpt1 = _scatter(eds[1][0], rows[1], zeros_s)
            out = _node_last(
                hh, mpt0, mpt1, xc, vel, cent,
                params['Wv1'][l], params['bv1'][l].reshape(1, HID),
                params['Wv2'][l], params['bv2'][l].reshape(1, 1))
        else:
            mpm0 = _scatter(eds[0][0], rows[0], zeros_s)
            mpt0 = _scatter(eds[0][1], rows[0], zeros_s)
            mpm1 = _scatter(eds[1][0], rows[1], zeros_s)
            mpt1 = _scatter(eds[1][1], rows[1], zeros_s)
            hh, xc, rtab, ctab = _node_mid(
                hh, mpm0, mpm1, mpt0, mpt1, xc, vel,
                params['Wv1'][l], params['bv1'][l].reshape(1, HID),
                params['Wv2'][l], params['bv2'][l].reshape(1, 1),
                params['Wn1'][l][0:HID, :], params['Wn1'][l][HID:, :],
                params['bn1'][l].reshape(1, HID),
                params['Wn2'][l], params['bn2'][l].reshape(1, HID),
                we1[l + 1, 0:HID, :], we1[l + 1, HID:2 * HID, :])
    return out


# speedup vs baseline: 2.2396x; 1.0030x over previous
import functools

import jax
import jax.numpy as jnp
from jax import lax
from jax.experimental import pallas as pl
from jax.experimental.pallas import tpu as pltpu
from jax.experimental.pallas import tpu_sc as plsc

N = 10000
E = 320000
HID = 128
HALF = 64
N_LAYERS = 4
N_POINTS = 5

NC = 2
NS = 16
NW = NC * NS
CK = 128
EH = E // 2
RPT8 = 624
NTAIL = N - NS * RPT8

DG = 256
DS = 128
XO = 128
VO = 131

BE = 1280
BN = 2000


def _sc_mesh():
    return plsc.VectorSubcoreMesh(core_axis_name="c", subcore_axis_name="s",
                                  num_cores=NC, num_subcores=NS)


def _silu(v):
    return v * jax.nn.sigmoid(v)


def _cross(ax, ay, az, bx, by, bz):
    return (ay * bz - az * by, az * bx - ax * bz, ax * by - ay * bx)


def _frame_cols(xr, xl):
    dx, dy, dz = xr[0] - xl[0], xr[1] - xl[1], xr[2] - xl[2]
    radial = dx * dx + dy * dy + dz * dz
    nrm = jnp.sqrt(radial) + 1.0
    cd = (dx / nrm, dy / nrm, dz / nrm)
    cx, cy, cz = _cross(xr[0], xr[1], xr[2], xl[0], xl[1], xl[2])
    cn = jnp.sqrt(cx * cx + cy * cy + cz * cz) + 1.0
    cc = (cx / cn, cy / cn, cz / cn)
    cv = _cross(cd[0], cd[1], cd[2], cc[0], cc[1], cc[2])
    return cd, cc, cv, radial


@functools.lru_cache(maxsize=None)
def _make_gather2(ne):
    nchunk = ne // CK
    iters = -(-nchunk // NW)

    @functools.partial(
        pl.kernel,
        out_type=(jax.ShapeDtypeStruct((ne, DG), jnp.float32),
                  jax.ShapeDtypeStruct((ne, DG), jnp.float32)),
        mesh=_sc_mesh(),
        scratch_types=[
            pltpu.VMEM((CK,), jnp.int32),
            pltpu.VMEM((CK,), jnp.int32),
            pltpu.VMEM((CK, DG), jnp.float32),
            pltpu.VMEM((CK, DG), jnp.float32),
            pltpu.SemaphoreType.DMA,
            pltpu.SemaphoreType.DMA,
        ],
    )
    def k(ta, tb, ia, ib, oa, ob, idx_a, idx_b, buf_a, buf_b, sem_a, sem_b):
        wid = lax.axis_index("s") * NC + lax.axis_index("c")

        def body(i, carry):
            cid = wid + i * NW

            @pl.when(cid < nchunk)
            def _():
                base = cid * CK
                pltpu.sync_copy(ia.at[pl.ds(base, CK)], idx_a)
                pltpu.sync_copy(ib.at[pl.ds(base, CK)], idx_b)
                ca = pltpu.async_copy(ta.at[idx_a], buf_a, sem_a)
                cb = pltpu.async_copy(tb.at[idx_b], buf_b, sem_b)
                ca.wait()
                cb.wait()
                pltpu.sync_copy(buf_a, oa.at[pl.ds(base, CK)])
                pltpu.sync_copy(buf_b, ob.at[pl.ds(base, CK)])

            return carry

        lax.fori_loop(0, iters, body, 0)

    return k


@functools.lru_cache(maxsize=None)
def _make_scatter(ne):
    nchunk = ne // CK
    iters = -(-nchunk // NW)

    @functools.partial(
        pl.kernel,
        out_type=jax.ShapeDtypeStruct((NC, N, DS), jnp.float32),
        mesh=_sc_mesh(),
        scratch_types=[
            pltpu.VMEM((CK,), jnp.int32),
            pltpu.VMEM((CK, DS), jnp.float32),
            pltpu.VMEM_SHARED((N, DS), jnp.float32),
        ],
    )
    def k(data, idx_hbm, zeros, out, idx_v, buf, acc):
        c = lax.axis_index("c")
        s = lax.axis_index("s")
        wid = s * NC + c
        ib = s * RPT8
        pltpu.sync_copy(zeros.at[pl.ds(ib, RPT8)], acc.at[pl.ds(ib, RPT8)])

        @pl.when(s == NS - 1)
        def _init_tail():
            pltpu.sync_copy(zeros.at[pl.ds(NS * RPT8, NTAIL)],
                            acc.at[pl.ds(NS * RPT8, NTAIL)])

        plsc.subcore_barrier()

        def body(i, carry):
            cid = wid + i * NW

            @pl.when(cid < nchunk)
            def _():
                base = cid * CK
                pltpu.sync_copy(idx_hbm.at[pl.ds(base, CK)], idx_v)
                pltpu.sync_copy(data.at[pl.ds(base, CK)], buf)
                pltpu.sync_copy(buf, acc.at[idx_v], add=True)

            return carry

        lax.fori_loop(0, iters, body, 0)
        plsc.subcore_barrier()
        pltpu.sync_copy(acc.at[pl.ds(ib, RPT8)], out.at[c, pl.ds(ib, RPT8)])

        @pl.when(s == NS - 1)
        def _drain_tail():
            pltpu.sync_copy(acc.at[pl.ds(NS * RPT8, NTAIL)],
                            out.at[c, pl.ds(NS * RPT8, NTAIL)])

    return k


def _gather2(ta, tb, ia, ib):
    return _make_gather2(ia.shape[0])(ta, tb, ia, ib)


def _scatter(data, idx, zeros):
    return _make_scatter(idx.shape[0])(data, idx, zeros)


def _full_spec(shape):
    nd = len(shape)
    return pl.BlockSpec(shape, lambda i, _n=nd: (0,) * _n)


def _center_body(x15_ref, m1_ref, m2_ref, xc15_ref, cent15_ref):
    x15 = x15_ref[...]
    cent = jnp.dot(jnp.dot(x15, m1_ref[...], preferred_element_type=jnp.float32),
                   m2_ref[...], preferred_element_type=jnp.float32)
    cent15_ref[...] = cent
    xc15_ref[...] = x15 - cent


def _center(x15, m1, m2):
    g = x15.shape[0]
    return pl.pallas_call(
        _center_body,
        grid=(1,),
        in_specs=[_full_spec((g, 15)), _full_spec((15, 3)), _full_spec((3, 15))],
        out_specs=[_full_spec((g, 15)), _full_spec((g, 15))],
        out_shape=[jax.ShapeDtypeStruct((g, 15), jnp.float32),
                   jax.ShapeDtypeStruct((g, 15), jnp.float32)],
    )(x15, m1, m2)


def _embed_body(h0_ref, xc_ref, vel_ref, wemb_ref, bemb_ref, wr_ref, wc_ref,
                h_ref, rtab_ref, ctab_ref):
    h = jnp.dot(h0_ref[...], wemb_ref[...], preferred_element_type=jnp.float32) \
        + bemb_ref[...]
    h_ref[...] = h
    xc = xc_ref[...]
    vel = vel_ref[...]
    z = jnp.zeros((h.shape[0], DG - VO - 3), jnp.float32)
    rtab_ref[...] = jnp.concatenate(
        [jnp.dot(h, wr_ref[...], preferred_element_type=jnp.float32), xc, vel, z],
        axis=1)
    ctab_ref[...] = jnp.concatenate(
        [jnp.dot(h, wc_ref[...], preferred_element_type=jnp.float32), xc, vel, z],
        axis=1)


def _embed(h0, xc, vel, wemb, bemb, wr, wc):
    grid = N // BN
    return pl.pallas_call(
        _embed_body,
        grid=(grid,),
        in_specs=[
            pl.BlockSpec((BN, HID), lambda i: (i, 0)),
            pl.BlockSpec((BN, 3), lambda i: (i, 0)),
            pl.BlockSpec((BN, 3), lambda i: (i, 0)),
            _full_spec((HID, HID)),
            _full_spec((1, HID)),
            _full_spec((HID, HID)),
            _full_spec((HID, HID)),
        ],
        out_specs=[
            pl.BlockSpec((BN, HID), lambda i: (i, 0)),
            pl.BlockSpec((BN, DG), lambda i: (i, 0)),
            pl.BlockSpec((BN, DG), lambda i: (i, 0)),
        ],
        out_shape=[jax.ShapeDtypeStruct((N, HID), jnp.float32),
                   jax.ShapeDtypeStruct((N, DG), jnp.float32),
                   jax.ShapeDtypeStruct((N, DG), jnp.float32)],
    )(h0, xc, vel, wemb, bemb, wr, wc)


def _cols3(arr, off):
    return (arr[:, off:off + 1], arr[:, off + 1:off + 2], arr[:, off + 2:off + 3])


def _edge0_body(gr_ref, gc_ref, ea_ref, wf1_ref, bf1_ref, wf2_ref, bf2_ref,
                wef_ref, wrad_ref, b1_ref, w2_ref, b2_ref, wc1_ref, bc1_ref,
                wc2_ref, m_ref, t_ref, ef_ref):
    gr = gr_ref[...]
    gc = gc_ref[...]
    xr = _cols3(gr, XO)
    vr = _cols3(gr, VO)
    xl = _cols3(gc, XO)
    vc = _cols3(gc, VO)
    cd, cc, cv, radial = _frame_cols(xr, xl)

    def proj(v):
        return (cd[0] * v[0] + cd[1] * v[1] + cd[2] * v[2],
                cc[0] * v[0] + cc[1] * v[1] + cc[2] * v[2],
                cv[0] * v[0] + cv[1] * v[1] + cv[2] * v[2])

    ci = proj(xr)
    cj = proj(xl)
    vi = proj(vr)
    vj = proj(vc)
    ni = jnp.sqrt(ci[0] * ci[0] + ci[1] * ci[1] + ci[2] * ci[2])
    nj = jnp.sqrt(cj[0] * cj[0] + cj[1] * cj[1] + cj[2] * cj[2])
    cos = (ci[0] * cj[0] + ci[1] * cj[1] + ci[2] * cj[2]) / (ni + 1e-05) / (nj + 1e-05)
    sin = jnp.sqrt(jnp.clip(1.0 - cos * cos, 0.0, None))
    feat = jnp.concatenate(
        [ea_ref[...], sin, cos, ci[0], ci[1], ci[2], cj[0], cj[1], cj[2],
         vi[0], vi[1], vi[2], vj[0], vj[1], vj[2]], axis=1)
    e1 = _silu(jnp.dot(feat, wf1_ref[...], preferred_element_type=jnp.float32)
               + bf1_ref[...])
    ef = _silu(jnp.dot(e1, wf2_ref[...], preferred_element_type=jnp.float32)
               + bf2_ref[...])
    ef_ref[...] = ef

    hs = gr[:, :HID] + gc[:, :HID]
    z1 = hs + radial * wrad_ref[...] + b1_ref[...] \
        + jnp.dot(ef, wef_ref[...], preferred_element_type=jnp.float32)
    a1 = _silu(z1)
    m = _silu(jnp.dot(a1, w2_ref[...], preferred_element_type=jnp.float32)
              + b2_ref[...])
    c1 = _silu(jnp.dot(m, wc1_ref[...], preferred_element_type=jnp.float32)
               + bc1_ref[...])
    coff = jnp.dot(c1, wc2_ref[...], preferred_element_type=jnp.float32)
    c0, c1_, c2 = coff[:, 0:1], coff[:, 1:2], coff[:, 2:3]
    tx = jnp.clip(cd[0] * c0 + cc[0] * c1_ + cv[0] * c2, -100.0, 100.0)
    ty = jnp.clip(cd[1] * c0 + cc[1] * c1_ + cv[1] * c2, -100.0, 100.0)
    tz = jnp.clip(cd[2] * c0 + cc[2] * c1_ + cv[2] * c2, -100.0, 100.0)
    one = jnp.ones_like(tx)
    pad = jnp.zeros((gr.shape[0], DS - 4), jnp.float32)
    m_ref[...] = m
    t_ref[...] = jnp.concatenate([tx, ty, tz, one, pad], axis=1)


def _edge0_mlp(gr, gc, ea, wf1, bf1, wf2, bf2,
               wef, wrad, b1, w2, b2, wc1, bc1, wc2):
    ne = gr.shape[0]
    grid = ne // BE
    return pl.pallas_call(
        _edge0_body,
        grid=(grid,),
        in_specs=[
            pl.BlockSpec((BE, DG), lambda i: (i, 0)),
            pl.BlockSpec((BE, DG), lambda i: (i, 0)),
            pl.BlockSpec((BE, 2), lambda i: (i, 0)),
            _full_spec((16, HALF)),
            _full_spec((1, HALF)),
            _full_spec((HALF, HALF)),
            _full_spec((1, HALF)),
            _full_spec((HALF, HID)),
            _full_spec((1, HID)),
            _full_spec((1, HID)),
            _full_spec((HID, HID)),
            _full_spec((1, HID)),
            _full_spec((HID, HID)),
            _full_spec((1, HID)),
            _full_spec((HID, 3)),
        ],
        out_specs=[pl.BlockSpec((BE, HID), lambda i: (i, 0)),
                   pl.BlockSpec((BE, DS), lambda i: (i, 0)),
                   pl.BlockSpec((BE, HALF), lambda i: (i, 0))],
        out_shape=[jax.ShapeDtypeStruct((ne, HID), jnp.float32),
                   jax.ShapeDtypeStruct((ne, DS), jnp.float32),
                   jax.ShapeDtypeStruct((ne, HALF), jnp.float32)],
    )(gr, gc, ea, wf1, bf1, wf2, bf2, wef, wrad, b1, w2, b2, wc1, bc1, wc2)


def _edge_body(last, gr_ref, gc_ref, ef_ref, wef_ref, wrad_ref, b1_ref,
               w2_ref, b2_ref, wc1_ref, bc1_ref, wc2_ref, *out_refs):
    gr = gr_ref[...]
    gc = gc_ref[...]
    hs = gr[:, :HID] + gc[:, :HID]
    xr = _cols3(gr, XO)
    xl = _cols3(gc, XO)
    cd, cc, cv, radial = _frame_cols(xr, xl)
    z1 = hs + radial * wrad_ref[...] + b1_ref[...] \
        + jnp.dot(ef_ref[...], wef_ref[...], preferred_element_type=jnp.float32)
    a1 = _silu(z1)
    m = _silu(jnp.dot(a1, w2_ref[...], preferred_element_type=jnp.float32)
              + b2_ref[...])
    c1 = _silu(jnp.dot(m, wc1_ref[...], preferred_element_type=jnp.float32)
               + bc1_ref[...])
    coff = jnp.dot(c1, wc2_ref[...], preferred_element_type=jnp.float32)
    c0, c1_, c2 = coff[:, 0:1], coff[:, 1:2], coff[:, 2:3]
    tx = jnp.clip(cd[0] * c0 + cc[0] * c1_ + cv[0] * c2, -100.0, 100.0)
    ty = jnp.clip(cd[1] * c0 + cc[1] * c1_ + cv[1] * c2, -100.0, 100.0)
    tz = jnp.clip(cd[2] * c0 + cc[2] * c1_ + cv[2] * c2, -100.0, 100.0)
    one = jnp.ones_like(tx)
    pad = jnp.zeros((gr.shape[0], DS - 4), jnp.float32)
    tvec = jnp.concatenate([tx, ty, tz, one, pad], axis=1)
    if last:
        out_refs[0][...] = tvec
    else:
        out_refs[0][...] = m
        out_refs[1][...] = tvec


def _edge_mlp(last, gr, gc, ef, wef, wrad, b1, w2, b2, wc1, bc1, wc2):
    ne = gr.shape[0]
    grid = ne // BE
    if last:
        out_specs = [pl.BlockSpec((BE, DS), lambda i: (i, 0))]
        out_shape = [jax.ShapeDtypeStruct((ne, DS), jnp.float32)]
    else:
        out_specs = [pl.BlockSpec((BE, HID), lambda i: (i, 0)),
                     pl.BlockSpec((BE, DS), lambda i: (i, 0))]
        out_shape = [jax.ShapeDtypeStruct((ne, HID), jnp.float32),
                     jax.ShapeDtypeStruct((ne, DS), jnp.float32)]
    return pl.pallas_call(
        functools.partial(_edge_body, last),
        grid=(grid,),
        in_specs=[
            pl.BlockSpec((BE, DG), lambda i: (i, 0)),
            pl.BlockSpec((BE, DG), lambda i: (i, 0)),
            pl.BlockSpec((BE, HALF), lambda i: (i, 0)),
            _full_spec((HALF, HID)),
            _full_spec((1, HID)),
            _full_spec((1, HID)),
            _full_spec((HID, HID)),
            _full_spec((1, HID)),
            _full_spec((HID, HID)),
            _full_spec((1, HID)),
            _full_spec((HID, 3)),
        ],
        out_specs=out_specs,
        out_shape=out_shape,
    )(gr, gc, ef, wef, wrad, b1, w2, b2, wc1, bc1, wc2)


def _node_mid_body(h_ref, mpm0_ref, mpm1_ref, mpt0_ref, mpt1_ref,
                   xc_ref, vel_ref,
                   wv1_ref, bv1_ref, wv2_ref, bv2_ref,
                   wn1h_ref, wn1g_ref, bn1_ref, wn2_ref, bn2_ref,
                   wrn_ref, wcn_ref,
                   hn_ref, xcn_ref, rtab_ref, ctab_ref):
    h = h_ref[...]
    mpm0 = mpm0_ref[...]
    mpm1 = mpm1_ref[...]
    mpt0 = mpt0_ref[...]
    mpt1 = mpt1_ref[...]
    hag = mpm0[0] + mpm0[1] + mpm1[0] + mpm1[1]
    tsum = mpt0[0] + mpt0[1] + mpt1[0] + mpt1[1]
    agg = tsum[:, 0:3]
    cnt = tsum[:, 3:4]
    xc = xc_ref[...] + agg / jnp.maximum(cnt, 1.0)
    vmul = jnp.dot(_silu(jnp.dot(h, wv1_ref[...], preferred_element_type=jnp.float32)
                         + bv1_ref[...]),
                   wv2_ref[...], preferred_element_type=jnp.float32) + bv2_ref[...]
    xc = xc + vmul * vel_ref[...]
    t = _silu(jnp.dot(h, wn1h_ref[...], preferred_element_type=jnp.float32)
              + jnp.dot(hag, wn1g_ref[...], preferred_element_type=jnp.float32)
              + bn1_ref[...])
    hn = h + jnp.dot(t, wn2_ref[...], preferred_element_type=jnp.float32) + bn2_ref[...]
    hn_ref[...] = hn
    xcn_ref[...] = xc
    z = jnp.zeros((h.shape[0], DG - XO - 3), jnp.float32)
    rtab_ref[...] = jnp.concatenate(
        [jnp.dot(hn, wrn_ref[...], preferred_element_type=jnp.float32), xc, z], axis=1)
    ctab_ref[...] = jnp.concatenate(
        [jnp.dot(hn, wcn_ref[...], preferred_element_type=jnp.float32), xc, z], axis=1)


def _node_mid(h, mpm0, mpm1, mpt0, mpt1, xc, vel, wv1, bv1, wv2, bv2,
              wn1h, wn1g, bn1, wn2, bn2, wrn, wcn):
    grid = N // BN
    return pl.pallas_call(
        _node_mid_body,
        grid=(grid,),
        in_specs=[
            pl.BlockSpec((BN, HID), lambda i: (i, 0)),
            pl.BlockSpec((NC, BN, HID), lambda i: (0, i, 0)),
            pl.BlockSpec((NC, BN, HID), lambda i: (0, i, 0)),
            pl.BlockSpec((NC, BN, DS), lambda i: (0, i, 0)),
            pl.BlockSpec((NC, BN, DS), lambda i: (0, i, 0)),
            pl.BlockSpec((BN, 3), lambda i: (i, 0)),
            pl.BlockSpec((BN, 3), lambda i: (i, 0)),
            _full_spec((HID, HID)),
            _full_spec((1, HID)),
            _full_spec((HID, 1)),
            _full_spec((1, 1)),
            _full_spec((HID, HID)),
            _full_spec((HID, HID)),
            _full_spec((1, HID)),
            _full_spec((HID, HID)),
            _full_spec((1, HID)),
            _full_spec((HID, HID)),
            _full_spec((HID, HID)),
        ],
        out_specs=[
            pl.BlockSpec((BN, HID), lambda i: (i, 0)),
            pl.BlockSpec((BN, 3), lambda i: (i, 0)),
            pl.BlockSpec((BN, DG), lambda i: (i, 0)),
            pl.BlockSpec((BN, DG), lambda i: (i, 0)),
        ],
        out_shape=[jax.ShapeDtypeStruct((N, HID), jnp.float32),
                   jax.ShapeDtypeStruct((N, 3), jnp.float32),
                   jax.ShapeDtypeStruct((N, DG), jnp.float32),
                   jax.ShapeDtypeStruct((N, DG), jnp.float32)],
    )(h, mpm0, mpm1, mpt0, mpt1, xc, vel, wv1, bv1, wv2, bv2,
      wn1h, wn1g, bn1, wn2, bn2, wrn, wcn)


def _node_last_body(h_ref, mpt0_ref, mpt1_ref, xc_ref, vel_ref, cent_ref,
                    wv1_ref, bv1_ref, wv2_ref, bv2_ref, out_ref):
    h = h_ref[...]
    mpt0 = mpt0_ref[...]
    mpt1 = mpt1_ref[...]
    tsum = mpt0[0] + mpt0[1] + mpt1[0] + mpt1[1]
    agg = tsum[:, 0:3]
    cnt = tsum[:, 3:4]
    xc = xc_ref[...] + agg / jnp.maximum(cnt, 1.0)
    vmul = jnp.dot(_silu(jnp.dot(h, wv1_ref[...], preferred_element_type=jnp.float32)
                         + bv1_ref[...]),
                   wv2_ref[...], preferred_element_type=jnp.float32) + bv2_ref[...]
    xc = xc + vmul * vel_ref[...]
    out_ref[...] = xc + cent_ref[...]


def _node_last(h, mpt0, mpt1, xc, vel, cent, wv1, bv1, wv2, bv2):
    grid = N // BN
    return pl.pallas_call(
        _node_last_body,
        grid=(grid,),
        in_specs=[
            pl.BlockSpec((BN, HID), lambda i: (i, 0)),
            pl.BlockSpec((NC, BN, DS), lambda i: (0, i, 0)),
            pl.BlockSpec((NC, BN, DS), lambda i: (0, i, 0)),
            pl.BlockSpec((BN, 3), lambda i: (i, 0)),
            pl.BlockSpec((BN, 3), lambda i: (i, 0)),
            pl.BlockSpec((BN, 3), lambda i: (i, 0)),
            _full_spec((HID, HID)),
            _full_spec((1, HID)),
            _full_spec((HID, 1)),
            _full_spec((1, 1)),
        ],
        out_specs=pl.BlockSpec((BN, 3), lambda i: (i, 0)),
        out_shape=jax.ShapeDtypeStruct((N, 3), jnp.float32),
    )(h, mpt0, mpt1, xc, vel, cent, wv1, bv1, wv2, bv2)


def kernel(h, x, vel, edge_attr, params, edges):
    row = edges[0]
    col = edges[1]
    f32 = jnp.float32

    m1 = jnp.zeros((15, 3), f32).at[jnp.arange(15), jnp.arange(15) % 3].set(1.0)
    m2 = jnp.zeros((3, 15), f32).at[jnp.arange(15) % 3, jnp.arange(15)].set(0.2)

    x15 = x.reshape(N // N_POINTS, 15)
    xc15, cent15 = _center(x15, m1, m2)
    xc = xc15.reshape(N, 3)
    cent = cent15.reshape(N, 3)

    we1 = params['We1']
    wr0 = we1[0, 0:HID, :]
    wc0 = we1[0, HID:2 * HID, :]
    hh, rtab, ctab = _embed(
        h, xc, vel, params['Wemb'], params['bemb'].reshape(1, HID), wr0, wc0)

    zeros_s = jnp.zeros((N, DS), f32)
    rows = (row[:EH], row[EH:])
    cols = (col[:EH], col[EH:])
    eas = (edge_attr[:EH], edge_attr[EH:])
    efs = [None, None]

    for l in range(N_LAYERS):
        last = l == N_LAYERS - 1
        eds = []
        for hf in range(2):
            gr, gc = _gather2(rtab, ctab, rows[hf], cols[hf])
            if l == 0:
                m0, t0, efs[hf] = _edge0_mlp(
                    gr, gc, eas[hf],
                    params['Wf1'], params['bf1'].reshape(1, HALF),
                    params['Wf2'], params['bf2'].reshape(1, HALF),
                    we1[l, 2 * HID + 1:, :], we1[l, 2 * HID:2 * HID + 1, :],
                    params['be1'][l].reshape(1, HID),
                    params['We2'][l], params['be2'][l].reshape(1, HID),
                    params['Wc1'][l], params['bc1'][l].reshape(1, HID),
                    params['Wc2'][l])
                eds.append((m0, t0))
            else:
                eds.append(_edge_mlp(
                    last, gr, gc, efs[hf],
                    we1[l, 2 * HID + 1:, :], we1[l, 2 * HID:2 * HID + 1, :],
                    params['be1'][l].reshape(1, HID),
                    params['We2'][l], params['be2'][l].reshape(1, HID),
                    params['Wc1'][l], params['bc1'][l].reshape(1, HID),
                    params['Wc2'][l]))
        if last:
            mpt0 = _scatter(eds[0][0], rows[0], zeros_s)
            mpt1 = _scatter(eds[1][0], rows[1], zeros_s)
            out = _node_last(
                hh, mpt0, mpt1, xc, vel, cent,
                params['Wv1'][l], params['bv1'][l].reshape(1, HID),
                params['Wv2'][l], params['bv2'][l].reshape(1, 1))
        else:
            mpm0 = _scatter(eds[0][0], rows[0], zeros_s)
            mpt0 = _scatter(eds[0][1], rows[0], zeros_s)
            mpm1 = _scatter(eds[1][0], rows[1], zeros_s)
            mpt1 = _scatter(eds[1][1], rows[1], zeros_s)
            hh, xc, rtab, ctab = _node_mid(
                hh, mpm0, mpm1, mpt0, mpt1, xc, vel,
                params['Wv1'][l], params['bv1'][l].reshape(1, HID),
                params['Wv2'][l], params['bv2'][l].reshape(1, 1),
                params['Wn1'][l][0:HID, :], params['Wn1'][l][HID:, :],
                params['bn1'][l].reshape(1, HID),
                params['Wn2'][l], params['bn2'][l].reshape(1, HID),
                we1[l + 1, 0:HID, :], we1[l + 1, HID:2 * HID, :])
    return out
